# Initial kernel scaffold; baseline (speedup 1.0000x reference)
import jax
import jax.numpy as jnp
from jax.experimental import pallas as pl


def kernel(flat, weight, cu_seqlens):
    raise NotImplementedError("write your pallas kernel here")


# speedup vs baseline: 2.2243x; 2.2243x over previous
import jax
import jax.numpy as jnp
from jax import lax
from jax.experimental import pallas as pl
from jax.experimental.pallas import tpu as pltpu
from jax.experimental.pallas import tpu_sc as plsc

DIM = 1024
T_ROWS = 8192
NSEG = 8
OUT_ROWS = T_ROWS + NSEG
NW = 32
PERW = T_ROWS // NW
S = 32
NCH = PERW // S
NG = S // 16


def _take(v, idx):
    dnums = lax.GatherDimensionNumbers(
        offset_dims=(), collapsed_slice_dims=(0,), start_index_map=(0,))
    return lax.gather(v, idx[:, None], dnums, slice_sizes=(1,),
                      mode=lax.GatherScatterMode.PROMISE_IN_BOUNDS)


def _allmax(v):
    lane = lax.iota(jnp.int32, 16)
    for sh in (8, 4, 2, 1):
        v = jnp.maximum(v, _take(v, lane ^ sh))
    return v


def _body(flat, w, cu16, out, idx0, idx1, tidx, buf0, buf1, tbuf, wv, cuv,
          fixref, tfixref, g0, g1, s0, s1, ts):
    cid = lax.axis_index("c")
    sid = lax.axis_index("s")
    wid = sid * 2 + cid
    base = wid * PERW

    pltpu.sync_copy(w, wv)
    pltpu.sync_copy(cu16, cuv)

    lane = lax.iota(jnp.int32, 16)
    nc = cuv[...] + lane
    ncs = [_take(nc, jnp.full((16,), j, jnp.int32)) for j in range(1, NSEG)]

    idx_refs = [idx0, idx1]
    bufs = [buf0, buf1]
    gsems = [g0, g1]
    ssems = [s0, s1]

    def compute_idx(ch):
        cbase = base + ch * S
        ref = idx_refs[ch % 2]
        for g in range(NG):
            pos = cbase + 16 * g + lane
            seg = jnp.zeros((16,), jnp.int32)
            isc = pos == 0
            for v in ncs:
                seg = seg + jnp.where(pos >= v, 1, 0)
                isc = isc | (pos == v)
            src = jnp.maximum(pos - seg - 1, 0)
            ref[pl.ds(16 * g, 16)] = src
            fixref[pl.ds(16 * (ch * NG + g), 16)] = _allmax(
                jnp.where(isc, pos, -1))

    def gather_start(ch):
        pltpu.make_async_copy(flat.at[idx_refs[ch % 2]], bufs[ch % 2],
                              gsems[ch % 2]).start()

    def gather_wait(ch):
        pltpu.make_async_copy(flat.at[idx_refs[ch % 2]], bufs[ch % 2],
                              gsems[ch % 2]).wait()

    def scatter_start(ch):
        pltpu.make_async_copy(bufs[ch % 2], out.at[pl.ds(base + ch * S, S)],
                              ssems[ch % 2]).start()

    def scatter_wait(ch):
        pltpu.make_async_copy(bufs[ch % 2], out.at[pl.ds(base + ch * S, S)],
                              ssems[ch % 2]).wait()

    compute_idx(0)
    gather_start(0)
    for i in range(NCH):
        if i + 1 < NCH:
            if i + 1 >= 2:
                scatter_wait(i - 1)
            compute_idx(i + 1)
            gather_start(i + 1)
        gather_wait(i)
        scatter_start(i)
    scatter_wait(NCH - 2)
    scatter_wait(NCH - 1)

    for gi in range(NCH * NG):
        f = fixref[pl.ds(16 * gi, 16)][0]

        @pl.when(f >= 0)
        def _(f=f):
            pltpu.sync_copy(wv, out.at[pl.ds(f, 1)])

    @pl.when(wid == NW - 1)
    def _():
        pos_raw = T_ROWS + lane
        pos = jnp.minimum(pos_raw, OUT_ROWS - 1)
        seg = jnp.zeros((16,), jnp.int32)
        isc = pos_raw < 0
        for v in ncs:
            seg = seg + jnp.where(pos >= v, 1, 0)
            isc = isc | (pos_raw == v)
        src = jnp.maximum(pos - seg - 1, 0)
        tidx[pl.ds(0, 16)] = src
        pltpu.make_async_copy(flat.at[tidx], tbuf, ts).start()
        pltpu.make_async_copy(flat.at[tidx], tbuf, ts).wait()
        pltpu.make_async_copy(tbuf.at[pl.ds(0, NSEG)],
                              out.at[pl.ds(T_ROWS, NSEG)], ts).start()
        pltpu.make_async_copy(tbuf.at[pl.ds(0, NSEG)],
                              out.at[pl.ds(T_ROWS, NSEG)], ts).wait()
        tfixref[pl.ds(0, 16)] = _allmax(jnp.where(isc, pos_raw, -1))
        tf = tfixref[pl.ds(0, 16)][0]

        @pl.when(tf >= 0)
        def _():
            pltpu.sync_copy(wv, out.at[pl.ds(tf, 1)])


def kernel(flat, weight, cu_seqlens):
    cu16 = jnp.zeros((16,), jnp.int32).at[:NSEG + 1].set(
        cu_seqlens.astype(jnp.int32))
    mesh = plsc.VectorSubcoreMesh(core_axis_name="c", subcore_axis_name="s")
    f = pl.kernel(
        _body,
        out_type=jax.ShapeDtypeStruct((OUT_ROWS, DIM), jnp.float32),
        mesh=mesh,
        scratch_types=[
            pltpu.VMEM((S,), jnp.int32),
            pltpu.VMEM((S,), jnp.int32),
            pltpu.VMEM((16,), jnp.int32),
            pltpu.VMEM((S, DIM), jnp.float32),
            pltpu.VMEM((S, DIM), jnp.float32),
            pltpu.VMEM((16, DIM), jnp.float32),
            pltpu.VMEM((1, DIM), jnp.float32),
            pltpu.VMEM((16,), jnp.int32),
            pltpu.VMEM((NCH * NG * 16,), jnp.int32),
            pltpu.VMEM((16,), jnp.int32),
            pltpu.SemaphoreType.DMA,
            pltpu.SemaphoreType.DMA,
            pltpu.SemaphoreType.DMA,
            pltpu.SemaphoreType.DMA,
            pltpu.SemaphoreType.DMA,
        ],
    )
    return f(flat, weight, cu16)


# speedup vs baseline: 2.2353x; 1.0049x over previous
import jax
import jax.numpy as jnp
from jax import lax
from jax.experimental import pallas as pl
from jax.experimental.pallas import tpu as pltpu
from jax.experimental.pallas import tpu_sc as plsc

DIM = 1024
T_ROWS = 8192
NSEG = 8
OUT_ROWS = T_ROWS + NSEG
NW = 32
PERW = T_ROWS // NW
S = 32
NCH = PERW // S
NG = S // 16


def _take(v, idx):
    dnums = lax.GatherDimensionNumbers(
        offset_dims=(), collapsed_slice_dims=(0,), start_index_map=(0,))
    return lax.gather(v, idx[:, None], dnums, slice_sizes=(1,),
                      mode=lax.GatherScatterMode.PROMISE_IN_BOUNDS)


def _allmax(v):
    lane = lax.iota(jnp.int32, 16)
    for sh in (8, 4, 2, 1):
        v = jnp.maximum(v, _take(v, lane ^ sh))
    return v


NSLOT = 3


def _body(flat, w, cu, out, idx0, idx1, idx2, tidx, buf0, buf1, buf2, tbuf,
          wv, cuv, fixref, tfixref, g0, g1, g2, s0, s1, s2, ts):
    cid = lax.axis_index("c")
    sid = lax.axis_index("s")
    wid = sid * 2 + cid
    base = wid * PERW

    pltpu.sync_copy(cu.at[pl.ds(0, NSEG)], cuv.at[pl.ds(0, NSEG)])

    lane = lax.iota(jnp.int32, 16)
    nc = cuv[...] + lane
    ncs = [_take(nc, jnp.full((16,), j, jnp.int32)) for j in range(1, NSEG)]

    idx_refs = [idx0, idx1, idx2]
    bufs = [buf0, buf1, buf2]
    gsems = [g0, g1, g2]
    ssems = [s0, s1, s2]

    def compute_idx(ch):
        cbase = base + ch * S
        ref = idx_refs[ch % NSLOT]
        for g in range(NG):
            pos = cbase + 16 * g + lane
            seg = jnp.zeros((16,), jnp.int32)
            isc = pos == 0
            for v in ncs:
                seg = seg + jnp.where(pos >= v, 1, 0)
                isc = isc | (pos == v)
            src = jnp.maximum(pos - seg - 1, 0)
            ref[pl.ds(16 * g, 16)] = src
            fixref[pl.ds(16 * (ch * NG + g), 16)] = _allmax(
                jnp.where(isc, pos, -1))

    def gather_start(ch):
        pltpu.make_async_copy(flat.at[idx_refs[ch % NSLOT]], bufs[ch % NSLOT],
                              gsems[ch % NSLOT]).start()

    def gather_wait(ch):
        pltpu.make_async_copy(flat.at[idx_refs[ch % NSLOT]], bufs[ch % NSLOT],
                              gsems[ch % NSLOT]).wait()

    def scatter_start(ch):
        pltpu.make_async_copy(bufs[ch % NSLOT],
                              out.at[pl.ds(base + ch * S, S)],
                              ssems[ch % NSLOT]).start()

    def scatter_wait(ch):
        pltpu.make_async_copy(bufs[ch % NSLOT],
                              out.at[pl.ds(base + ch * S, S)],
                              ssems[ch % NSLOT]).wait()

    compute_idx(0)
    gather_start(0)
    compute_idx(1)
    gather_start(1)
    for i in range(NCH):
        gather_wait(i)
        scatter_start(i)
        nxt = i + 2
        if nxt < NCH:
            if nxt - NSLOT >= 0:
                scatter_wait(nxt - NSLOT)
            compute_idx(nxt)
            gather_start(nxt)
    for ch in range(max(0, NCH - NSLOT), NCH):
        scatter_wait(ch)

    pltpu.sync_copy(w, wv)
    for gi in range(NCH * NG):
        f = fixref[pl.ds(16 * gi, 16)][0]

        @pl.when(f >= 0)
        def _(f=f):
            pltpu.sync_copy(wv, out.at[pl.ds(f, 1)])

    @pl.when(wid == NW - 1)
    def _():
        pos_raw = T_ROWS + lane
        pos = jnp.minimum(pos_raw, OUT_ROWS - 1)
        seg = jnp.zeros((16,), jnp.int32)
        isc = pos_raw < 0
        for v in ncs:
            seg = seg + jnp.where(pos >= v, 1, 0)
            isc = isc | (pos_raw == v)
        src = jnp.maximum(pos - seg - 1, 0)
        tidx[pl.ds(0, 16)] = src
        pltpu.make_async_copy(flat.at[tidx], tbuf, ts).start()
        pltpu.make_async_copy(flat.at[tidx], tbuf, ts).wait()
        pltpu.make_async_copy(tbuf.at[pl.ds(0, NSEG)],
                              out.at[pl.ds(T_ROWS, NSEG)], ts).start()
        pltpu.make_async_copy(tbuf.at[pl.ds(0, NSEG)],
                              out.at[pl.ds(T_ROWS, NSEG)], ts).wait()
        tfixref[pl.ds(0, 16)] = _allmax(jnp.where(isc, pos_raw, -1))
        tf = tfixref[pl.ds(0, 16)][0]

        @pl.when(tf >= 0)
        def _():
            pltpu.sync_copy(wv, out.at[pl.ds(tf, 1)])


def kernel(flat, weight, cu_seqlens):
    mesh = plsc.VectorSubcoreMesh(core_axis_name="c", subcore_axis_name="s")
    f = pl.kernel(
        _body,
        out_type=jax.ShapeDtypeStruct((OUT_ROWS, DIM), jnp.float32),
        mesh=mesh,
        scratch_types=[
            pltpu.VMEM((S,), jnp.int32),
            pltpu.VMEM((S,), jnp.int32),
            pltpu.VMEM((S,), jnp.int32),
            pltpu.VMEM((16,), jnp.int32),
            pltpu.VMEM((S, DIM), jnp.float32),
            pltpu.VMEM((S, DIM), jnp.float32),
            pltpu.VMEM((S, DIM), jnp.float32),
            pltpu.VMEM((16, DIM), jnp.float32),
            pltpu.VMEM((1, DIM), jnp.float32),
            pltpu.VMEM((16,), jnp.int32),
            pltpu.VMEM((NCH * NG * 16,), jnp.int32),
            pltpu.VMEM((16,), jnp.int32),
            pltpu.SemaphoreType.DMA,
            pltpu.SemaphoreType.DMA,
            pltpu.SemaphoreType.DMA,
            pltpu.SemaphoreType.DMA,
            pltpu.SemaphoreType.DMA,
            pltpu.SemaphoreType.DMA,
            pltpu.SemaphoreType.DMA,
        ],
    )
    return f(flat, weight, cu_seqlens)
